# baseline (speedup 1.0000x reference)
import functools

import jax
import jax.numpy as jnp
from jax import lax
from jax.experimental import pallas as pl
from jax.experimental.pallas import tpu as pltpu
from jax.experimental.pallas import tpu_sc as plsc

R = 262144
NCODES = 64
CODE = 4
N = 2097152
QUARTER = R // 4

_TC_BLOCK = 2048
_TC_GRID = QUARTER // _TC_BLOCK


def _code_body(x0, x1, x2, x3, out_ref):
    word = None
    for q, ref in enumerate((x0, x1, x2, x3)):
        x = ref[...]
        m = jnp.max(x, axis=0)
        it = lax.broadcasted_iota(jnp.int32, x.shape, 0)
        a = jnp.min(jnp.where(x == m[None, :], it, NCODES), axis=0)
        part = a << (8 * q)
        word = part if word is None else word | part
    out_ref[...] = word


_codes_tc = pl.pallas_call(
    _code_body,
    grid=(_TC_GRID,),
    in_specs=[
        pl.BlockSpec((NCODES, _TC_BLOCK), lambda i, q=q: (0, q * _TC_GRID + i))
        for q in range(4)
    ],
    out_specs=pl.BlockSpec((_TC_BLOCK,), lambda i: (i,)),
    out_shape=jax.ShapeDtypeStruct((QUARTER,), jnp.int32),
)

_NW = 32
_PER_W = N // _NW
_CHUNK = 4096
_NCH = _PER_W // _CHUNK
_SCALE = 0.5 * (R - 1)

_mesh = plsc.VectorSubcoreMesh(core_axis_name="c", subcore_axis_name="s")


@functools.partial(
    pl.kernel,
    mesh=_mesh,
    out_type=jax.ShapeDtypeStruct((N * CODE,), jnp.float32),
    compiler_params=pltpu.CompilerParams(needs_layout_passes=False),
    scratch_types=[
        pltpu.VMEM((QUARTER,), jnp.int32),
        pltpu.VMEM((NCODES * CODE,), jnp.float32),
        pltpu.VMEM((NCODES,), jnp.float32),
        pltpu.VMEM((NCODES,), jnp.float32),
        pltpu.VMEM((NCODES,), jnp.float32),
        pltpu.VMEM((NCODES,), jnp.float32),
        pltpu.VMEM((_CHUNK,), jnp.float32),
        pltpu.VMEM((_CHUNK,), jnp.float32),
        pltpu.VMEM((_CHUNK * CODE,), jnp.float32),
        pltpu.VMEM((_CHUNK * CODE,), jnp.float32),
        pltpu.SemaphoreType.DMA,
        pltpu.SemaphoreType.DMA,
        pltpu.SemaphoreType.DMA,
        pltpu.SemaphoreType.DMA,
    ],
)
def _interp_sc(coords_hbm, packed_hbm, cb_hbm, out_hbm,
               packed_v, cbflat_v, cb0, cb1, cb2, cb3,
               cbuf0, cbuf1, obuf0, obuf1, sin0, sin1, sout0, sout1):
    wid = lax.axis_index("s") * 2 + lax.axis_index("c")
    base = wid * _PER_W

    pltpu.async_copy(coords_hbm.at[pl.ds(base, _CHUNK)], cbuf0, sin0)
    pltpu.async_copy(coords_hbm.at[pl.ds(base + _CHUNK, _CHUNK)], cbuf1, sin1)
    pltpu.sync_copy(packed_hbm, packed_v)
    pltpu.sync_copy(cb_hbm, cbflat_v)

    cbcols = (cb0, cb1, cb2, cb3)
    for k in range(CODE):
        for s in range(NCODES // 16):
            cbcols[k][pl.ds(s * 16, 16)] = cbflat_v[pl.ds(k * NCODES + s * 16, 16)]

    def compute(cbuf, obuf):
        @plsc.parallel_loop(0, _CHUNK // 16, unroll=8)
        def _(s):
            j0 = s * 16
            c = (cbuf[pl.ds(j0, 16)] + 1.0) * _SCALE
            li = jnp.minimum(c.astype(jnp.int32), R - 2)
            w = c - li.astype(jnp.float32)
            ri = li + 1
            pw_l = plsc.load_gather(packed_v, [li & 0xFFFF])
            code_l = (pw_l >> ((li >> 16) << 3)) & 63
            pw_r = plsc.load_gather(packed_v, [ri & 0xFFFF])
            code_r = (pw_r >> ((ri >> 16) << 3)) & 63
            off0 = ((j0 >> 7) << 9) + (j0 & 127)
            for k in range(CODE):
                lv = plsc.load_gather(cbcols[k], [code_l])
                rv = plsc.load_gather(cbcols[k], [code_r])
                obuf[pl.ds(off0 + k * 128, 16)] = lv + w * (rv - lv)

    bufs = ((cbuf0, obuf0, sin0, sout0), (cbuf1, obuf1, sin1, sout1))

    def pair(i2, carry):
        for b in range(2):
            cbuf, obuf, sin, sout = bufs[b]
            ci = i2 * 2 + b
            off = base + ci * _CHUNK
            pltpu.make_async_copy(
                coords_hbm.at[pl.ds(base, _CHUNK)], cbuf, sin).wait()

            @pl.when(ci >= 2)
            def _():
                pltpu.make_async_copy(
                    obuf, out_hbm.at[pl.ds(base * CODE, _CHUNK * CODE)], sout
                ).wait()

            compute(cbuf, obuf)
            pltpu.async_copy(
                obuf, out_hbm.at[pl.ds(off * CODE, _CHUNK * CODE)], sout)

            @pl.when(ci + 2 < _NCH)
            def _():
                pltpu.async_copy(
                    coords_hbm.at[pl.ds(off + 2 * _CHUNK, _CHUNK)], cbuf, sin)
        return carry

    lax.fori_loop(0, _NCH // 2, pair, 0)

    for b in range(2):
        cbuf, obuf, sin, sout = bufs[b]
        pltpu.make_async_copy(
            obuf, out_hbm.at[pl.ds(base * CODE, _CHUNK * CODE)], sout).wait()


def kernel(coords, codebook, indices):
    packed = _codes_tc(*([indices.T] * 4))
    flat = _interp_sc(coords, packed, codebook.T.reshape(-1))
    return flat.reshape(N // 128, CODE, 128).transpose(0, 2, 1).reshape(N, CODE)


# speedup vs baseline: 1.3351x; 1.3351x over previous
import functools

import jax
import jax.numpy as jnp
from jax import lax
from jax.experimental import pallas as pl
from jax.experimental.pallas import tpu as pltpu
from jax.experimental.pallas import tpu_sc as plsc

R = 262144
NCODES = 64
CODE = 4
N = 2097152
QUARTER = R // 4

_TC_BLOCK = 2048
_TC_GRID = QUARTER // _TC_BLOCK


def _code_body(x0, x1, x2, x3, out_ref):
    word = None
    for q, ref in enumerate((x0, x1, x2, x3)):
        x = ref[...]
        m = jnp.max(x, axis=0)
        it = lax.broadcasted_iota(jnp.int32, x.shape, 0)
        a = jnp.min(jnp.where(x == m[None, :], it, NCODES), axis=0)
        part = a << (8 * q)
        word = part if word is None else word | part
    out_ref[...] = word


_codes_tc = pl.pallas_call(
    _code_body,
    grid=(_TC_GRID,),
    in_specs=[
        pl.BlockSpec((NCODES, _TC_BLOCK), lambda i, q=q: (0, q * _TC_GRID + i))
        for q in range(4)
    ],
    out_specs=pl.BlockSpec((_TC_BLOCK,), lambda i: (i,)),
    out_shape=jax.ShapeDtypeStruct((QUARTER,), jnp.int32),
)

_NW = 32
_PER_W = N // _NW
_CHUNK = 4096
_NCH = _PER_W // _CHUNK
_SCALE = 0.5 * (R - 1)

_mesh = plsc.VectorSubcoreMesh(core_axis_name="c", subcore_axis_name="s")


@functools.partial(
    pl.kernel,
    mesh=_mesh,
    out_type=jax.ShapeDtypeStruct((N * CODE,), jnp.float32),
    compiler_params=pltpu.CompilerParams(needs_layout_passes=False),
    scratch_types=[
        pltpu.VMEM((QUARTER,), jnp.int32),
        pltpu.VMEM((NCODES * CODE,), jnp.float32),
        pltpu.VMEM((NCODES * 16,), jnp.float32),
        pltpu.VMEM((NCODES * 16,), jnp.float32),
        pltpu.VMEM((NCODES * 16,), jnp.float32),
        pltpu.VMEM((NCODES * 16,), jnp.float32),
        pltpu.VMEM((_CHUNK,), jnp.float32),
        pltpu.VMEM((_CHUNK,), jnp.float32),
        pltpu.VMEM((_CHUNK * CODE,), jnp.float32),
        pltpu.VMEM((_CHUNK * CODE,), jnp.float32),
        pltpu.SemaphoreType.DMA,
        pltpu.SemaphoreType.DMA,
        pltpu.SemaphoreType.DMA,
        pltpu.SemaphoreType.DMA,
    ],
)
def _interp_sc(coords_hbm, packed_hbm, cb_hbm, out_hbm,
               packed_v, cbflat_v, cb0, cb1, cb2, cb3,
               cbuf0, cbuf1, obuf0, obuf1, sin0, sin1, sout0, sout1):
    wid = lax.axis_index("s") * 2 + lax.axis_index("c")
    base = wid * _PER_W

    pltpu.async_copy(coords_hbm.at[pl.ds(base, _CHUNK)], cbuf0, sin0)
    pltpu.async_copy(coords_hbm.at[pl.ds(base + _CHUNK, _CHUNK)], cbuf1, sin1)
    pltpu.sync_copy(packed_hbm, packed_v)
    pltpu.sync_copy(cb_hbm, cbflat_v)

    it = lax.iota(jnp.int32, 16)
    it16 = it << 4
    cbcols = (cb0, cb1, cb2, cb3)
    for k in range(CODE):
        for s in range(NCODES // 16):
            v = cbflat_v[pl.ds(k * NCODES + s * 16, 16)]
            for rep in range(16):
                plsc.store_scatter(cbcols[k], [it16 + (s * 256 + rep)], v)

    def compute(cbuf, obuf):
        @plsc.parallel_loop(0, _CHUNK // 16, unroll=4)
        def _(s):
            j0 = s * 16
            c = (cbuf[pl.ds(j0, 16)] + 1.0) * _SCALE
            li = jnp.minimum(c.astype(jnp.int32), R - 2)
            w = c - li.astype(jnp.float32)
            ri = li + 1
            pw_l = plsc.load_gather(packed_v, [li & 0xFFFF])
            idx_l = (((pw_l >> ((li >> 16) << 3)) & 63) << 4) + it
            pw_r = plsc.load_gather(packed_v, [ri & 0xFFFF])
            idx_r = (((pw_r >> ((ri >> 16) << 3)) & 63) << 4) + it
            off0 = ((j0 >> 7) << 9) + (j0 & 127)
            for k in range(CODE):
                lv = plsc.load_gather(cbcols[k], [idx_l])
                rv = plsc.load_gather(cbcols[k], [idx_r])
                obuf[pl.ds(off0 + k * 128, 16)] = lv + w * (rv - lv)

    bufs = ((cbuf0, obuf0, sin0, sout0), (cbuf1, obuf1, sin1, sout1))

    def pair(i2, carry):
        for b in range(2):
            cbuf, obuf, sin, sout = bufs[b]
            ci = i2 * 2 + b
            off = base + ci * _CHUNK
            pltpu.make_async_copy(
                coords_hbm.at[pl.ds(base, _CHUNK)], cbuf, sin).wait()

            @pl.when(ci >= 2)
            def _():
                pltpu.make_async_copy(
                    obuf, out_hbm.at[pl.ds(base * CODE, _CHUNK * CODE)], sout
                ).wait()

            compute(cbuf, obuf)
            pltpu.async_copy(
                obuf, out_hbm.at[pl.ds(off * CODE, _CHUNK * CODE)], sout)

            @pl.when(ci + 2 < _NCH)
            def _():
                pltpu.async_copy(
                    coords_hbm.at[pl.ds(off + 2 * _CHUNK, _CHUNK)], cbuf, sin)
        return carry

    lax.fori_loop(0, _NCH // 2, pair, 0)

    for b in range(2):
        cbuf, obuf, sin, sout = bufs[b]
        pltpu.make_async_copy(
            obuf, out_hbm.at[pl.ds(base * CODE, _CHUNK * CODE)], sout).wait()


def kernel(coords, codebook, indices):
    packed = _codes_tc(*([indices.T] * 4))
    flat = _interp_sc(coords, packed, codebook.T.reshape(-1))
    return flat.reshape(N // 128, CODE, 128).transpose(0, 2, 1).reshape(N, CODE)


# speedup vs baseline: 1.3823x; 1.0354x over previous
import functools

import jax
import jax.numpy as jnp
from jax import lax
from jax.experimental import pallas as pl
from jax.experimental.pallas import tpu as pltpu
from jax.experimental.pallas import tpu_sc as plsc

R = 262144
NCODES = 64
CODE = 4
N = 2097152
QUARTER = R // 4

_TC_BLOCK = 2048
_QSTRIDE = QUARTER // _TC_BLOCK
_W_TC = 40960
_W_SC = QUARTER - _W_TC


def _code_body(x0, x1, x2, x3, out_ref):
    word = None
    for q, ref in enumerate((x0, x1, x2, x3)):
        x = ref[...]
        m = jnp.max(x, axis=0)
        it = lax.broadcasted_iota(jnp.int32, x.shape, 0)
        a = jnp.min(jnp.where(x == m[None, :], it, NCODES), axis=0)
        part = a << (8 * q)
        word = part if word is None else word | part
    out_ref[...] = word


_codes_tc = pl.pallas_call(
    _code_body,
    grid=(_W_TC // _TC_BLOCK,),
    in_specs=[
        pl.BlockSpec((NCODES, _TC_BLOCK), lambda i, q=q: (0, q * _QSTRIDE + i))
        for q in range(4)
    ],
    out_specs=pl.BlockSpec((_TC_BLOCK,), lambda i: (i,)),
    out_shape=jax.ShapeDtypeStruct((_W_TC,), jnp.int32),
)

_NW = 32
_WPT = _W_SC // _NW
_KB = _WPT // 128
_mesh = plsc.VectorSubcoreMesh(core_axis_name="c", subcore_axis_name="s")


@functools.partial(
    pl.kernel,
    mesh=_mesh,
    out_type=jax.ShapeDtypeStruct((_W_SC,), jnp.int32),
    compiler_params=pltpu.CompilerParams(needs_layout_passes=False),
    scratch_types=[
        pltpu.VMEM((8192,), jnp.float32),
        pltpu.VMEM((8192,), jnp.float32),
        pltpu.VMEM((_WPT,), jnp.int32),
        pltpu.SemaphoreType.DMA,
        pltpu.SemaphoreType.DMA,
    ],
)
def _codes_sc(ind_hbm, outw_hbm, lb0, lb1, ow, s0, s1):
    wid = lax.axis_index("s") * 2 + lax.axis_index("c")
    wbase = _W_TC + wid * _WPT
    rt0 = wbase >> 7

    def issue(chunk, lb, sem):
        b = chunk // _KB
        k = chunk % _KB
        rt = b * 512 + rt0 + k
        for ct in range(8):
            pltpu.async_copy(
                ind_hbm.at[pl.ds((ct * 2048 + rt) * 1024, 1024)],
                lb.at[pl.ds(ct * 1024, 1024)], sem)

    def drain(lb, sem):
        for ct in range(8):
            pltpu.make_async_copy(
                ind_hbm.at[pl.ds(0, 1024)],
                lb.at[pl.ds(ct * 1024, 1024)], sem).wait()

    def compute(chunk, lb):
        b = chunk // _KB
        k = chunk % _KB

        @plsc.parallel_loop(0, 8, unroll=2)
        def _(g):
            rl0 = g * 16
            m = lb[pl.ds(rl0, 16)]
            idx = jnp.zeros((16,), jnp.int32)
            for c in range(1, NCODES):
                off = (c >> 3) * 1024 + (c & 7) * 128
                v = lb[pl.ds(off + rl0, 16)]
                gt = v > m
                idx = jnp.where(gt, c, idx)
                m = jnp.where(gt, v, m)
            word = idx << (b * 8)
            loc = k * 128 + rl0
            prev = ow[pl.ds(loc, 16)]
            ow[pl.ds(loc, 16)] = jnp.where(b == 0, word, prev | word)

    nch = 4 * _KB
    issue(0, lb0, s0)
    issue(1, lb1, s1)

    def pairs(i2, carry):
        for par, (lb, sem) in enumerate(((lb0, s0), (lb1, s1))):
            chunk = i2 * 2 + par
            drain(lb, sem)
            compute(chunk, lb)

            @pl.when(chunk + 2 < nch)
            def _():
                issue(chunk + 2, lb, sem)
        return carry

    lax.fori_loop(0, nch // 2, pairs, 0)
    pltpu.sync_copy(ow, outw_hbm.at[pl.ds(wid * _WPT, _WPT)])

_PER_W = N // _NW
_CHUNK = 4096
_NCH = _PER_W // _CHUNK
_SCALE = 0.5 * (R - 1)


@functools.partial(
    pl.kernel,
    mesh=_mesh,
    out_type=jax.ShapeDtypeStruct((N * CODE,), jnp.float32),
    compiler_params=pltpu.CompilerParams(needs_layout_passes=False),
    scratch_types=[
        pltpu.VMEM((QUARTER,), jnp.int32),
        pltpu.VMEM((NCODES * CODE,), jnp.float32),
        pltpu.VMEM((NCODES * 16,), jnp.float32),
        pltpu.VMEM((NCODES * 16,), jnp.float32),
        pltpu.VMEM((NCODES * 16,), jnp.float32),
        pltpu.VMEM((NCODES * 16,), jnp.float32),
        pltpu.VMEM((_CHUNK,), jnp.float32),
        pltpu.VMEM((_CHUNK,), jnp.float32),
        pltpu.VMEM((_CHUNK * CODE,), jnp.float32),
        pltpu.VMEM((_CHUNK * CODE,), jnp.float32),
        pltpu.SemaphoreType.DMA,
        pltpu.SemaphoreType.DMA,
        pltpu.SemaphoreType.DMA,
        pltpu.SemaphoreType.DMA,
    ],
)
def _interp_sc(coords_hbm, ptc_hbm, psc_hbm, cb_hbm, out_hbm,
               packed_v, cbflat_v, cb0, cb1, cb2, cb3,
               cbuf0, cbuf1, obuf0, obuf1, sin0, sin1, sout0, sout1):
    wid = lax.axis_index("s") * 2 + lax.axis_index("c")
    base = wid * _PER_W

    pltpu.async_copy(coords_hbm.at[pl.ds(base, _CHUNK)], cbuf0, sin0)
    pltpu.async_copy(coords_hbm.at[pl.ds(base + _CHUNK, _CHUNK)], cbuf1, sin1)
    pltpu.sync_copy(ptc_hbm, packed_v.at[pl.ds(0, _W_TC)])
    pltpu.sync_copy(psc_hbm, packed_v.at[pl.ds(_W_TC, _W_SC)])
    pltpu.sync_copy(cb_hbm, cbflat_v)

    it = lax.iota(jnp.int32, 16)
    it16 = it << 4
    cbcols = (cb0, cb1, cb2, cb3)
    for k in range(CODE):
        for s in range(NCODES // 16):
            v = cbflat_v[pl.ds(k * NCODES + s * 16, 16)]
            for rep in range(16):
                plsc.store_scatter(cbcols[k], [it16 + (s * 256 + rep)], v)

    def compute(cbuf, obuf):
        @plsc.parallel_loop(0, _CHUNK // 16, unroll=4)
        def _(s):
            j0 = s * 16
            c = (cbuf[pl.ds(j0, 16)] + 1.0) * _SCALE
            li = jnp.minimum(c.astype(jnp.int32), R - 2)
            w = c - li.astype(jnp.float32)
            ri = li + 1
            pw_l = plsc.load_gather(packed_v, [li & 0xFFFF])
            idx_l = (((pw_l >> ((li >> 16) << 3)) & 63) << 4) + it
            pw_r = plsc.load_gather(packed_v, [ri & 0xFFFF])
            idx_r = (((pw_r >> ((ri >> 16) << 3)) & 63) << 4) + it
            off0 = ((j0 >> 7) << 9) + (j0 & 127)
            for k in range(CODE):
                lv = plsc.load_gather(cbcols[k], [idx_l])
                rv = plsc.load_gather(cbcols[k], [idx_r])
                obuf[pl.ds(off0 + k * 128, 16)] = lv + w * (rv - lv)

    bufs = ((cbuf0, obuf0, sin0, sout0), (cbuf1, obuf1, sin1, sout1))

    def pair(i2, carry):
        for b in range(2):
            cbuf, obuf, sin, sout = bufs[b]
            ci = i2 * 2 + b
            off = base + ci * _CHUNK
            pltpu.make_async_copy(
                coords_hbm.at[pl.ds(base, _CHUNK)], cbuf, sin).wait()

            @pl.when(ci >= 2)
            def _():
                pltpu.make_async_copy(
                    obuf, out_hbm.at[pl.ds(base * CODE, _CHUNK * CODE)], sout
                ).wait()

            compute(cbuf, obuf)
            pltpu.async_copy(
                obuf, out_hbm.at[pl.ds(off * CODE, _CHUNK * CODE)], sout)

            @pl.when(ci + 2 < _NCH)
            def _():
                pltpu.async_copy(
                    coords_hbm.at[pl.ds(off + 2 * _CHUNK, _CHUNK)], cbuf, sin)
        return carry

    lax.fori_loop(0, _NCH // 2, pair, 0)

    for b in range(2):
        cbuf, obuf, sin, sout = bufs[b]
        pltpu.make_async_copy(
            obuf, out_hbm.at[pl.ds(base * CODE, _CHUNK * CODE)], sout).wait()


def kernel(coords, codebook, indices):
    iT = indices.T
    ind_flat = iT.reshape(8, 8, 2048, 128).transpose(0, 2, 1, 3).reshape(-1)
    packed_tc = _codes_tc(*([iT] * 4))
    packed_sc = _codes_sc(ind_flat)
    flat = _interp_sc(coords, packed_tc, packed_sc, codebook.T.reshape(-1))
    return flat.reshape(N // 128, CODE, 128).transpose(0, 2, 1).reshape(N, CODE)
